# baseline (speedup 1.0000x reference)
import dataclasses
import functools

import jax
import jax.numpy as jnp
from jax import lax
from jax.experimental import pallas as pl
from jax.experimental.pallas import tpu as pltpu
from jax.experimental.pallas import tpu_sc as plsc

B = 16384
D = 32
NC = 2
NS = 16
NW = NC * NS
BPW = B // NW
CHUNK = 128
NCHUNK = BPW // CHUNK
L = 16
NG = BPW // L


def _mf_body(uid_hbm, mid_hbm, p_hbm, q_hbm, bu_hbm, bi_hbm, out_hbm,
             uid_v, mid_v, pu_v, qi_v, bu_v, bi_v, out_v, sem):
    wid = lax.axis_index("s") * NC + lax.axis_index("c")
    base = wid * BPW

    pltpu.sync_copy(uid_hbm.at[pl.ds(base, BPW)], uid_v)
    pltpu.sync_copy(mid_hbm.at[pl.ds(base, BPW)], mid_v)

    copies = []
    for c in range(NCHUNK):
        sl = pl.ds(c * CHUNK, CHUNK)
        copies.append(pltpu.async_copy(p_hbm.at[uid_v.at[sl]], pu_v.at[sl], sem))
        copies.append(pltpu.async_copy(q_hbm.at[mid_v.at[sl]], qi_v.at[sl], sem))
        copies.append(pltpu.async_copy(bu_hbm.at[uid_v.at[sl]], bu_v.at[sl], sem))
        copies.append(pltpu.async_copy(bi_hbm.at[mid_v.at[sl]], bi_v.at[sl], sem))
    for cp in copies:
        cp.wait()

    lane = lax.iota(jnp.int32, L)

    @pl.loop(0, NG)
    def _(g):
        rows = g * L + lane
        acc = bu_v[pl.ds(g * L, L)] + bi_v[pl.ds(g * L, L)]
        for k in range(D):
            kk = jnp.full((L,), k, jnp.int32)
            acc = acc + (plsc.load_gather(pu_v, [rows, kk]) *
                         plsc.load_gather(qi_v, [rows, kk]))
        out_v[pl.ds(g * L, L)] = acc

    pltpu.sync_copy(out_v, out_hbm.at[pl.ds(base, BPW)])


@jax.jit
def kernel(user_id, movie_id, P, Q, b_u, b_i):
    user_id = user_id.astype(jnp.int32)
    movie_id = movie_id.astype(jnp.int32)
    mesh = plsc.VectorSubcoreMesh(core_axis_name="c", subcore_axis_name="s",
                                  num_cores=NC, num_subcores=NS)
    cp = pltpu.CompilerParams()
    if "needs_layout_passes" in pltpu.CompilerParams.__dataclass_fields__:
        cp = dataclasses.replace(cp, needs_layout_passes=False)
    if "use_tc_tiling_on_sc" in pltpu.CompilerParams.__dataclass_fields__:
        cp = dataclasses.replace(cp, use_tc_tiling_on_sc=False)
    mf = pl.kernel(
        _mf_body,
        out_type=jax.ShapeDtypeStruct((B,), jnp.float32),
        mesh=mesh,
        scratch_types=[
            pltpu.VMEM((BPW,), jnp.int32),
            pltpu.VMEM((BPW,), jnp.int32),
            pltpu.VMEM((BPW, D), jnp.float32),
            pltpu.VMEM((BPW, D), jnp.float32),
            pltpu.VMEM((BPW,), jnp.float32),
            pltpu.VMEM((BPW,), jnp.float32),
            pltpu.VMEM((BPW,), jnp.float32),
            pltpu.SemaphoreType.DMA,
        ],
        compiler_params=cp,
    )
    return mf(user_id, movie_id, P, Q, b_u.reshape(-1), b_i.reshape(-1))
